# baseline (speedup 1.0000x reference)
import functools

import jax
import jax.numpy as jnp
from jax import lax
from jax.experimental import pallas as pl
from jax.experimental.pallas import tpu as pltpu
from jax.experimental.pallas import tpu_sc as plsc

D = 64
NC = 2
NS = 16
NW = NC * NS
IPS = 128
NBUF = 4
L = 16


@functools.partial(jax.jit, static_argnums=(2,))
def _emb(words_flat, table_flat, b_total):
    table = table_flat.reshape(table_flat.shape[0] // D, D)
    units_per_w = b_total // (NW * IPS)
    nbeats = units_per_w // NBUF
    h_units = 4096 // IPS

    @functools.partial(
        pl.kernel,
        mesh=plsc.VectorSubcoreMesh(core_axis_name="c", subcore_axis_name="s"),
        out_type=jax.ShapeDtypeStruct((200, 8, 32, 8, IPS), jnp.float32),
        scratch_types=[
            pltpu.VMEM((units_per_w * IPS,), jnp.int32),
            pltpu.VMEM((NBUF, IPS, D), jnp.float32),
            pltpu.VMEM((NBUF, 8, 8, IPS), jnp.float32),
            pltpu.SemaphoreType.DMA((NBUF,)),
            pltpu.SemaphoreType.DMA((NBUF,)),
        ],
        compiler_params=pltpu.CompilerParams(
            use_tc_tiling_on_sc=False, needs_layout_passes=False
        ),
    )
    def k(words_hbm, table_hbm, out_hbm, idx_v, rows_v, trans_v, gsem, osem):
        wid = lax.axis_index("s") * NC + lax.axis_index("c")
        u_base = wid * units_per_w

        pltpu.sync_copy(
            words_hbm.at[pl.ds(u_base * IPS, units_per_w * IPS)], idx_v
        )

        row_iotas = [
            lax.iota(jnp.int32, L) + (cb * L) for cb in range(IPS // L)
        ]

        def gather_copy(u, s):
            return pltpu.make_async_copy(
                table_hbm.at[idx_v.at[pl.ds(u * IPS, IPS)]],
                rows_v.at[s],
                gsem.at[s],
            )

        def out_copy(u, s):
            gu = u_base + u
            h = gu // h_units
            bt = gu % h_units
            return pltpu.make_async_copy(
                trans_v.at[s],
                out_hbm.at[h, :, bt],
                osem.at[s],
            )

        def transpose_unit(s):
            def ft_body(ft, carry):
                for r in range(8):
                    col = jnp.full((L,), ft * 8 + r, jnp.int32)
                    for cb in range(IPS // L):
                        g = plsc.load_gather(
                            rows_v.at[s], [row_iotas[cb], col]
                        )
                        trans_v[s, ft, r, pl.ds(cb * L, L)] = g
                return carry

            lax.fori_loop(0, 8, ft_body, 0)

        for s in range(NBUF):
            gather_copy(s, s).start()

        def beat(t, carry):
            for s in range(NBUF):
                u = t * NBUF + s
                gather_copy(u, s).wait()

                @pl.when(t > 0)
                def _drain():
                    out_copy(u, s).wait()

                transpose_unit(s)
                out_copy(u, s).start()

                @pl.when(t < nbeats - 1)
                def _prefetch():
                    gather_copy(u + NBUF, s).start()

            return carry

        lax.fori_loop(0, nbeats, beat, 0)

        for s in range(NBUF):
            out_copy((nbeats - 1) * NBUF + s, s).wait()

    return k(words_flat, table)


def kernel(words, table):
    b, h = words.shape
    wt = words.T.reshape(b * h)
    out5 = _emb(wt, table.reshape(-1), b * h)
    return out5.transpose(2, 4, 0, 1, 3).reshape(b, h, D)


# speedup vs baseline: 1.8484x; 1.8484x over previous
import functools

import jax
import jax.numpy as jnp
from jax import lax
from jax.experimental import pallas as pl
from jax.experimental.pallas import tpu as pltpu
from jax.experimental.pallas import tpu_sc as plsc

D = 64
NC = 2
NS = 16
NW = NC * NS
IPS = 128
NBUF = 4
L = 16


@functools.partial(jax.jit, static_argnums=(2,))
def _emb(words_flat, table_flat, b_total):
    table = table_flat.reshape(table_flat.shape[0] // D, D)
    units_per_w = b_total // (NW * IPS)
    nbeats = units_per_w // NBUF
    h_units = 4096 // IPS

    @functools.partial(
        pl.kernel,
        mesh=plsc.VectorSubcoreMesh(core_axis_name="c", subcore_axis_name="s"),
        out_type=jax.ShapeDtypeStruct((200, 8, 32, 8, IPS), jnp.float32),
        scratch_types=[
            pltpu.VMEM((units_per_w * IPS,), jnp.int32),
            pltpu.VMEM((NBUF, IPS, D), jnp.float32),
            pltpu.VMEM((NBUF, 8, 8, IPS), jnp.float32),
            pltpu.SemaphoreType.DMA((NBUF,)),
            pltpu.SemaphoreType.DMA((NBUF,)),
        ],
        compiler_params=pltpu.CompilerParams(
            use_tc_tiling_on_sc=False, needs_layout_passes=False
        ),
    )
    def k(words_hbm, table_hbm, out_hbm, idx_v, rows_v, trans_v, gsem, osem):
        wid = lax.axis_index("s") * NC + lax.axis_index("c")
        u_base = wid * units_per_w

        pltpu.sync_copy(
            words_hbm.at[pl.ds(u_base * IPS, units_per_w * IPS)], idx_v
        )

        row_iotas = [
            lax.iota(jnp.int32, L) + (cb * L) for cb in range(IPS // L)
        ]

        def gather_copy(u, s):
            return pltpu.make_async_copy(
                table_hbm.at[idx_v.at[pl.ds(u * IPS, IPS)]],
                rows_v.at[s],
                gsem.at[s],
            )

        def out_copy(u, s):
            gu = u_base + u
            h = gu // h_units
            bt = gu % h_units
            return pltpu.make_async_copy(
                trans_v.at[s],
                out_hbm.at[h, :, bt],
                osem.at[s],
            )

        lvec = lax.iota(jnp.int32, L)

        def transpose_unit(s):
            def d_body(d, carry):
                fd = (lvec + d) & (L - 1)
                for f0b in range(D // L):
                    f_vec = fd + f0b * L
                    fhi = f_vec >> 3
                    flo = f_vec & 7
                    for c0b in range(IPS // L):
                        g = plsc.load_gather(
                            rows_v.at[s], [row_iotas[c0b], f_vec]
                        )
                        plsc.store_scatter(
                            trans_v.at[s], [fhi, flo, row_iotas[c0b]], g
                        )
                return carry

            lax.fori_loop(0, L, d_body, 0)

        for s in range(NBUF):
            gather_copy(s, s).start()

        def beat(t, carry):
            for s in range(NBUF):
                u = t * NBUF + s
                gather_copy(u, s).wait()

                @pl.when(t > 0)
                def _drain():
                    out_copy(u, s).wait()

                transpose_unit(s)
                out_copy(u, s).start()

                @pl.when(t < nbeats - 1)
                def _prefetch():
                    gather_copy(u + NBUF, s).start()

            return carry

        lax.fori_loop(0, nbeats, beat, 0)

        for s in range(NBUF):
            out_copy((nbeats - 1) * NBUF + s, s).wait()

    return k(words_flat, table)


def kernel(words, table):
    b, h = words.shape
    wt = words.T.reshape(b * h)
    out5 = _emb(wt, table.reshape(-1), b * h)
    return out5.transpose(2, 4, 0, 1, 3).reshape(b, h, D)


# speedup vs baseline: 2.2739x; 1.2302x over previous
import functools

import jax
import jax.numpy as jnp
from jax import lax
from jax.experimental import pallas as pl
from jax.experimental.pallas import tpu as pltpu
from jax.experimental.pallas import tpu_sc as plsc

D = 64
NC = 2
NS = 16
NW = NC * NS
IPS = 128
NBUF = 4
L = 16


@functools.partial(jax.jit, static_argnums=(2,))
def _emb(words_flat, table_flat, b_total):
    table = table_flat.reshape(table_flat.shape[0] // D, D)
    units_per_w = b_total // (NW * IPS)
    nbeats = units_per_w // NBUF
    h_units = 4096 // IPS

    @functools.partial(
        pl.kernel,
        mesh=plsc.VectorSubcoreMesh(core_axis_name="c", subcore_axis_name="s"),
        out_type=jax.ShapeDtypeStruct((200, 8, 32, 8, IPS), jnp.float32),
        scratch_types=[
            pltpu.VMEM((units_per_w * IPS,), jnp.int32),
            pltpu.VMEM((NBUF, IPS, D), jnp.float32),
            pltpu.VMEM((NBUF, 8, 8, IPS), jnp.float32),
            pltpu.SemaphoreType.DMA((NBUF,)),
            pltpu.SemaphoreType.DMA((NBUF,)),
        ],
        compiler_params=pltpu.CompilerParams(
            use_tc_tiling_on_sc=False, needs_layout_passes=False
        ),
    )
    def k(words_hbm, table_hbm, out_hbm, idx_v, rows_v, trans_v, gsem, osem):
        wid = lax.axis_index("s") * NC + lax.axis_index("c")
        u_base = wid * units_per_w

        pltpu.sync_copy(
            words_hbm.at[pl.ds(u_base * IPS, units_per_w * IPS)], idx_v
        )

        row_iotas = [
            lax.iota(jnp.int32, L) + (cb * L) for cb in range(IPS // L)
        ]

        def gather_copy(u, s):
            return pltpu.make_async_copy(
                table_hbm.at[idx_v.at[pl.ds(u * IPS, IPS)]],
                rows_v.at[s],
                gsem.at[s],
            )

        def out_copy(u, s):
            gu = u_base + u
            h = gu // h_units
            bt = gu % h_units
            return pltpu.make_async_copy(
                trans_v.at[s],
                out_hbm.at[h, :, bt],
                osem.at[s],
            )

        lvec = lax.iota(jnp.int32, L)

        def transpose_unit(s):
            def d_body(d4, carry):
                for dd in range(4):
                    fd = (lvec + (d4 * 4 + dd)) & (L - 1)
                    for f0b in range(D // L):
                        f_vec = fd + f0b * L
                        fhi = f_vec >> 3
                        flo = f_vec & 7
                        for c0b in range(IPS // L):
                            g = plsc.load_gather(
                                rows_v.at[s], [row_iotas[c0b], f_vec]
                            )
                            plsc.store_scatter(
                                trans_v.at[s], [fhi, flo, row_iotas[c0b]], g
                            )
                return carry

            lax.fori_loop(0, L // 4, d_body, 0)

        for s in range(NBUF):
            gather_copy(s, s).start()

        def beat(t, carry):
            for s in range(NBUF):
                u = t * NBUF + s
                gather_copy(u, s).wait()

                @pl.when(t > 0)
                def _drain():
                    out_copy(u, s).wait()

                transpose_unit(s)
                out_copy(u, s).start()

                @pl.when(t < nbeats - 1)
                def _prefetch():
                    gather_copy(u + NBUF, s).start()

            return carry

        lax.fori_loop(0, nbeats, beat, 0)

        for s in range(NBUF):
            out_copy((nbeats - 1) * NBUF + s, s).wait()

    return k(words_flat, table)


def kernel(words, table):
    b, h = words.shape
    wt = words.T.reshape(b * h)
    out5 = _emb(wt, table.reshape(-1), b * h)
    return out5.transpose(2, 4, 0, 1, 3).reshape(b, h, D)
